# baseline (speedup 1.0000x reference)
import functools

import jax
import jax.numpy as jnp
from jax import lax
from jax.experimental import pallas as pl
from jax.experimental.pallas import tpu as pltpu
from jax.experimental.pallas import tpu_sc as plsc

_T = 1024
_NB = 128
_CH = 128
_GB = 4


def _sc_gather(y, idx_pad, e_pad):
  _, d = y.shape
  info = plsc.get_sparse_core_info()
  nc, ns = info.num_cores, info.num_subcores
  nw = nc * ns
  cpw = e_pad // (nw * _CH)
  rounds = cpw // _GB
  nchunks = e_pad // _CH

  idx2 = idx_pad.reshape(nw, cpw, _CH)
  mesh = plsc.VectorSubcoreMesh(core_axis_name="c", subcore_axis_name="s")

  @functools.partial(
      pl.kernel,
      mesh=mesh,
      out_type=jax.ShapeDtypeStruct((nchunks, _CH, d), jnp.int32),
      scratch_types=[
          pltpu.VMEM((cpw, _CH), jnp.int32),
          pltpu.VMEM((_GB, _CH, d), jnp.int32),
          pltpu.SemaphoreType.DMA,
          pltpu.SemaphoreType.DMA,
      ],
      compiler_params=pltpu.CompilerParams(use_tc_tiling_on_sc=False),
  )
  def gk(y_hbm, idx_hbm, out_hbm, idx_v, rows_v, sem_g, sem_s):
    wid = lax.axis_index("s") * nc + lax.axis_index("c")
    pltpu.sync_copy(idx_hbm.at[wid], idx_v)
    chunk0 = wid * cpw

    def rnd(j, carry):
      @pl.when(j > 0)
      def _():
        pltpu.make_async_copy(
            rows_v, out_hbm.at[pl.ds(chunk0 + (j - 1) * _GB, _GB)],
            sem_s).wait()

      handles = []
      for b in range(_GB):
        handles.append(pltpu.async_copy(
            y_hbm.at[idx_v.at[j * _GB + b]], rows_v.at[b], sem_g))
      for hh in handles:
        hh.wait()
      pltpu.async_copy(
          rows_v, out_hbm.at[pl.ds(chunk0 + j * _GB, _GB)], sem_s)
      return carry

    lax.fori_loop(0, rounds, rnd, 0)
    pltpu.make_async_copy(
        rows_v, out_hbm.at[pl.ds(chunk0 + (rounds - 1) * _GB, _GB)],
        sem_s).wait()

  return gk(y, idx2)


def _tc_body(start_ref, nchunk_ref, rs1_ref, rs2_ref, y_ref, w1a_ref,
             w1b_ref, b1_ref, w2_ref, b2_ref, rep_ref, out_ref,
             rep_buf, s_acc, sem):
  nb = pl.program_id(0)
  e_start = start_ref[nb]
  nch = nchunk_ref[nb]

  def dma(i):
    slot = lax.rem(i, 2)
    e0 = pl.multiple_of(e_start + i * _T, 16)
    return pltpu.make_async_copy(
        rep_ref.at[pl.ds(e0, _T)], rep_buf.at[slot], sem.at[slot])

  bblk = jnp.dot(y_ref[...].astype(jnp.bfloat16),
                 w1b_ref[...].astype(jnp.bfloat16),
                 preferred_element_type=jnp.float32) + b1_ref[...]
  wcat = jnp.concatenate(
      [w1a_ref[...], bblk], axis=0).astype(jnp.bfloat16)

  rs1 = rs1_ref[0]
  rs2 = rs2_ref[0]

  s_acc[...] = jnp.zeros_like(s_acc)

  @pl.when(nch > 0)
  def _():
    dma(0).start()

  def chunk(i, carry):
    @pl.when(i + 1 < nch)
    def _():
      dma(i + 1).start()
    dma(i).wait()
    e0 = e_start + i * _T
    eg = e0 + lax.broadcasted_iota(jnp.int32, (_T, _NB), 0)
    oh = jnp.logical_and(eg >= rs1, eg < rs2).astype(jnp.bfloat16)
    x = jnp.concatenate([rep_buf[lax.rem(i, 2)], oh], axis=1)
    a = jnp.dot(x, wcat, preferred_element_type=jnp.float32)
    hid = 0.5 * a * (1.0 + lax.erf(a * 0.7071067811865476))
    s_acc[...] += lax.dot_general(
        oh, hid.astype(jnp.bfloat16), (((0,), (0,)), ((), ())),
        preferred_element_type=jnp.float32)
    return carry

  lax.fori_loop(0, nch, chunk, 0)

  cnt = (rs2 - rs1).astype(jnp.float32).reshape(_NB, 1)
  out_ref[...] = jnp.dot(
      s_acc[...].astype(jnp.bfloat16), w2_ref[...].astype(jnp.bfloat16),
      preferred_element_type=jnp.float32) + cnt * b2_ref[...]


def kernel(y, neighbors_index, neighbors_row_splits, W1, b1, W2, b2):
  n, d = y.shape
  e = neighbors_index.shape[0]
  h = W1.shape[1]
  nblk = -(-n // _NB)
  n_pad = nblk * _NB

  info = plsc.get_sparse_core_info()
  nw = info.num_cores * info.num_subcores
  align = nw * _CH * _GB
  e_pad = -(-(e + _T) // align) * align

  idx_pad = jnp.concatenate(
      [neighbors_index.astype(jnp.int32),
       jnp.zeros((e_pad - e,), jnp.int32)])
  y_packed = lax.bitcast_convert_type(
      y.astype(jnp.bfloat16).reshape(n, d // 2, 2), jnp.int32)
  rep32 = _sc_gather(y_packed, idx_pad, e_pad)
  rep = lax.bitcast_convert_type(rep32, jnp.bfloat16).reshape(e_pad, d)

  rs = neighbors_row_splits.astype(jnp.int32)
  rs_pad = jnp.concatenate([rs, jnp.full((n_pad - n,), e, jnp.int32)])
  jj = jnp.arange(n_pad)
  rs1 = rs_pad[jj].reshape(nblk, 1, _NB)
  rs2 = rs_pad[jj + 1].reshape(nblk, 1, _NB)
  bb = jnp.arange(nblk)
  starts = (rs_pad[bb * _NB] // 16) * 16
  ends = rs_pad[(bb + 1) * _NB]
  nchunks = (ends - starts + _T - 1) // _T

  y_pad = jnp.concatenate([y, jnp.zeros((n_pad - n, d), y.dtype)])
  w1a = W1[:d]
  w1b = W1[d:]
  b1r = b1.reshape(1, h)
  b2r = b2.reshape(1, d)

  grid_spec = pltpu.PrefetchScalarGridSpec(
      num_scalar_prefetch=2,
      grid=(nblk,),
      in_specs=[
          pl.BlockSpec((1, 1, _NB), lambda i, *_: (i, 0, 0)),
          pl.BlockSpec((1, 1, _NB), lambda i, *_: (i, 0, 0)),
          pl.BlockSpec((_NB, d), lambda i, *_: (i, 0)),
          pl.BlockSpec((d, h), lambda i, *_: (0, 0)),
          pl.BlockSpec((d, h), lambda i, *_: (0, 0)),
          pl.BlockSpec((1, h), lambda i, *_: (0, 0)),
          pl.BlockSpec((h, d), lambda i, *_: (0, 0)),
          pl.BlockSpec((1, d), lambda i, *_: (0, 0)),
          pl.BlockSpec(memory_space=pltpu.MemorySpace.HBM),
      ],
      out_specs=pl.BlockSpec((_NB, d), lambda i, *_: (i, 0)),
      scratch_shapes=[
          pltpu.VMEM((2, _T, d), jnp.bfloat16),
          pltpu.VMEM((_NB, h), jnp.float32),
          pltpu.SemaphoreType.DMA((2,)),
      ],
  )
  out = pl.pallas_call(
      _tc_body,
      grid_spec=grid_spec,
      out_shape=jax.ShapeDtypeStruct((n_pad, d), jnp.float32),
      compiler_params=pltpu.CompilerParams(
          dimension_semantics=("arbitrary",)),
  )(starts, nchunks, rs1, rs2, y_pad, w1a, w1b, b1r, W2, b2r, rep)
  return out[:n]


# speedup vs baseline: 1.5544x; 1.5544x over previous
import functools

import jax
import jax.numpy as jnp
from jax import lax
from jax.experimental import pallas as pl
from jax.experimental.pallas import tpu as pltpu
from jax.experimental.pallas import tpu_sc as plsc

_T = 1024
_NB = 128
_CH = 128
_GB = 4


def _sc_gather(y, idx_pad, e_pad):
  _, d = y.shape
  info = plsc.get_sparse_core_info()
  nc, ns = info.num_cores, info.num_subcores
  nw = nc * ns
  cpw = e_pad // (nw * _CH)
  rounds = cpw // _GB
  nchunks = e_pad // _CH

  idx2 = idx_pad.reshape(nw, cpw, _CH)
  mesh = plsc.VectorSubcoreMesh(core_axis_name="c", subcore_axis_name="s")

  @functools.partial(
      pl.kernel,
      mesh=mesh,
      out_type=jax.ShapeDtypeStruct((nchunks, _CH, d), jnp.float32),
      scratch_types=[
          pltpu.VMEM((cpw, _CH), jnp.int32),
          pltpu.VMEM((_GB, _CH, d), jnp.float32),
          pltpu.SemaphoreType.DMA,
          pltpu.SemaphoreType.DMA,
      ],
  )
  def gk(y_hbm, idx_hbm, out_hbm, idx_v, rows_v, sem_g, sem_s):
    wid = lax.axis_index("s") * nc + lax.axis_index("c")
    pltpu.sync_copy(idx_hbm.at[wid], idx_v)
    chunk0 = wid * cpw

    def rnd(j, carry):
      @pl.when(j > 0)
      def _():
        pltpu.make_async_copy(
            rows_v, out_hbm.at[pl.ds(chunk0 + (j - 1) * _GB, _GB)],
            sem_s).wait()

      handles = []
      for b in range(_GB):
        handles.append(pltpu.async_copy(
            y_hbm.at[idx_v.at[j * _GB + b]], rows_v.at[b], sem_g))
      for hh in handles:
        hh.wait()
      pltpu.async_copy(
          rows_v, out_hbm.at[pl.ds(chunk0 + j * _GB, _GB)], sem_s)
      return carry

    lax.fori_loop(0, rounds, rnd, 0)
    pltpu.make_async_copy(
        rows_v, out_hbm.at[pl.ds(chunk0 + (rounds - 1) * _GB, _GB)],
        sem_s).wait()

  return gk(y, idx2)


def _tc_body(start_ref, nchunk_ref, rs1_ref, rs2_ref, y_ref, w1a_ref,
             w1b_ref, b1_ref, w2_ref, b2_ref, rep_ref, out_ref,
             rep_buf, s_acc, sem):
  nb = pl.program_id(0)
  e_start = start_ref[nb]
  nch = nchunk_ref[nb]

  def dma(i):
    slot = lax.rem(i, 2)
    e0 = pl.multiple_of(e_start + i * _T, 16)
    return pltpu.make_async_copy(
        rep_ref.at[pl.ds(e0, _T)], rep_buf.at[slot], sem.at[slot])

  bblk = jnp.dot(y_ref[...].astype(jnp.bfloat16),
                 w1b_ref[...].astype(jnp.bfloat16),
                 preferred_element_type=jnp.float32) + b1_ref[...]
  wcat = jnp.concatenate(
      [w1a_ref[...], bblk], axis=0).astype(jnp.bfloat16)

  rs1 = rs1_ref[0]
  rs2 = rs2_ref[0]

  s_acc[...] = jnp.zeros_like(s_acc)

  @pl.when(nch > 0)
  def _():
    dma(0).start()

  def chunk(i, carry):
    @pl.when(i + 1 < nch)
    def _():
      dma(i + 1).start()
    dma(i).wait()
    e0 = e_start + i * _T
    eg = e0 + lax.broadcasted_iota(jnp.int32, (_T, _NB), 0)
    oh = jnp.logical_and(eg >= rs1, eg < rs2).astype(jnp.bfloat16)
    x = jnp.concatenate(
        [rep_buf[lax.rem(i, 2)].astype(jnp.bfloat16), oh], axis=1)
    a = jnp.dot(x, wcat, preferred_element_type=jnp.float32)
    hid = 0.5 * a * (1.0 + lax.erf(a * 0.7071067811865476))
    s_acc[...] += lax.dot_general(
        oh, hid.astype(jnp.bfloat16), (((0,), (0,)), ((), ())),
        preferred_element_type=jnp.float32)
    return carry

  lax.fori_loop(0, nch, chunk, 0)

  cnt = (rs2 - rs1).astype(jnp.float32).reshape(_NB, 1)
  out_ref[...] = jnp.dot(
      s_acc[...].astype(jnp.bfloat16), w2_ref[...].astype(jnp.bfloat16),
      preferred_element_type=jnp.float32) + cnt * b2_ref[...]


def kernel(y, neighbors_index, neighbors_row_splits, W1, b1, W2, b2):
  n, d = y.shape
  e = neighbors_index.shape[0]
  h = W1.shape[1]
  nblk = -(-n // _NB)
  n_pad = nblk * _NB

  info = plsc.get_sparse_core_info()
  nw = info.num_cores * info.num_subcores
  align = nw * _CH * _GB
  e_pad = -(-(e + _T) // align) * align

  idx_pad = jnp.concatenate(
      [neighbors_index.astype(jnp.int32),
       jnp.zeros((e_pad - e,), jnp.int32)])
  rep = _sc_gather(y, idx_pad, e_pad).reshape(e_pad, d)

  rs = neighbors_row_splits.astype(jnp.int32)
  rs_pad = jnp.concatenate([rs, jnp.full((n_pad - n,), e, jnp.int32)])
  jj = jnp.arange(n_pad)
  rs1 = rs_pad[jj].reshape(nblk, 1, _NB)
  rs2 = rs_pad[jj + 1].reshape(nblk, 1, _NB)
  bb = jnp.arange(nblk)
  starts = (rs_pad[bb * _NB] // 16) * 16
  ends = rs_pad[(bb + 1) * _NB]
  nchunks = (ends - starts + _T - 1) // _T

  y_pad = jnp.concatenate([y, jnp.zeros((n_pad - n, d), y.dtype)])
  w1a = W1[:d]
  w1b = W1[d:]
  b1r = b1.reshape(1, h)
  b2r = b2.reshape(1, d)

  grid_spec = pltpu.PrefetchScalarGridSpec(
      num_scalar_prefetch=2,
      grid=(nblk,),
      in_specs=[
          pl.BlockSpec((1, 1, _NB), lambda i, *_: (i, 0, 0)),
          pl.BlockSpec((1, 1, _NB), lambda i, *_: (i, 0, 0)),
          pl.BlockSpec((_NB, d), lambda i, *_: (i, 0)),
          pl.BlockSpec((d, h), lambda i, *_: (0, 0)),
          pl.BlockSpec((d, h), lambda i, *_: (0, 0)),
          pl.BlockSpec((1, h), lambda i, *_: (0, 0)),
          pl.BlockSpec((h, d), lambda i, *_: (0, 0)),
          pl.BlockSpec((1, d), lambda i, *_: (0, 0)),
          pl.BlockSpec(memory_space=pltpu.MemorySpace.HBM),
      ],
      out_specs=pl.BlockSpec((_NB, d), lambda i, *_: (i, 0)),
      scratch_shapes=[
          pltpu.VMEM((2, _T, d), jnp.float32),
          pltpu.VMEM((_NB, h), jnp.float32),
          pltpu.SemaphoreType.DMA((2,)),
      ],
  )
  out = pl.pallas_call(
      _tc_body,
      grid_spec=grid_spec,
      out_shape=jax.ShapeDtypeStruct((n_pad, d), jnp.float32),
      compiler_params=pltpu.CompilerParams(
          dimension_semantics=("arbitrary",)),
  )(starts, nchunks, rs1, rs2, y_pad, w1a, w1b, b1r, W2, b2r, rep)
  return out[:n]
